# Initial kernel scaffold; baseline (speedup 1.0000x reference)
import jax
import jax.numpy as jnp
from jax.experimental import pallas as pl


def kernel(pred, target, crop_masks):
    raise NotImplementedError("write your pallas kernel here")


# speedup vs baseline: 2.9672x; 2.9672x over previous
import jax
import jax.numpy as jnp
from jax import lax
from jax.experimental import pallas as pl

_H = 256
_W = 256
_NEG = -3.0e38


def _roll(x, sh, axis):
    n = x.shape[axis]
    sh %= n
    if sh == 0:
        return x
    if axis == 1:
        return jnp.concatenate([x[:, n - sh:], x[:, :n - sh]], axis=1)
    return jnp.concatenate([x[n - sh:, :], x[:n - sh, :]], axis=0)


def _lovasz_body(pred_ref, targ_ref, mask_ref, out_ref):
    p = pred_ref[0]
    t = targ_ref[0]
    m = mask_ref[0]

    row = lax.broadcasted_iota(jnp.int32, (_H, _W), 0)
    col = lax.broadcasted_iota(jnp.int32, (_H, _W), 1)

    signs = 2.0 * t - 1.0
    err = 1.0 - p * signs
    err = jnp.where(m > 0.0, err, _NEG)
    tm = t * m

    bits = lax.bitcast_convert_type(err, jnp.int32)
    key = jnp.where(bits < 0, bits ^ 0x7FFFFFFF, bits)
    key = (key & -2) | tm.astype(jnp.int32)

    for klog in range(1, 17):
        if klog < 8:
            desc = ((col >> klog) & 1) == 0
        else:
            desc = ((row >> (klog - 8)) & 1) == 0
        for j in range(klog - 1, -1, -1):
            if j < 8:
                s = 1 << j
                low = ((col >> j) & 1) == 0
                pr = jnp.where(low, _roll(key, -s, 1), _roll(key, s, 1))
            else:
                sr = 1 << (j - 8)
                low = ((row >> (j - 8)) & 1) == 0
                pr = jnp.where(low, _roll(key, -sr, 0), _roll(key, sr, 0))
            take_max = low == desc
            key = jnp.where(take_max, jnp.maximum(key, pr),
                            jnp.minimum(key, pr))

    gt = (key & 1).astype(jnp.float32)
    kb = key & -2
    ebits = jnp.where(kb < 0, kb ^ 0x7FFFFFFF, kb)
    es = lax.bitcast_convert_type(ebits, jnp.float32)
    relu = jnp.maximum(es, 0.0)

    cs = gt
    for sh in (1, 2, 4, 8, 16, 32, 64, 128):
        cs = cs + jnp.where(col >= sh, _roll(cs, sh, 1), 0.0)
    rowtot = jnp.broadcast_to(cs[:, _W - 1:_W], (_H, _W))
    ex = rowtot
    for sh in (1, 2, 4, 8, 16, 32, 64, 128):
        ex = ex + jnp.where(row >= sh, _roll(ex, sh, 0), 0.0)
    ex = ex - rowtot
    c_incl = cs + ex

    g_tot = jnp.sum(gt)
    jf = (row * _W + col + 1).astype(jnp.float32)
    jac = jf / (g_tot + jf - c_incl)

    nr = _roll(relu, -1, 1)
    ra0 = _roll(relu, -1, 0)
    col0 = jnp.broadcast_to(ra0[:, 0:1], (_H, _W))
    nr = jnp.where(col == _W - 1, col0, nr)
    nr = jnp.where((col == _W - 1) & (row == _H - 1), 0.0, nr)

    out_ref[0] = jnp.sum(jac * (relu - nr)).reshape(1, 1)


def kernel(pred, target, crop_masks):
    b, n, h, w = pred.shape
    k = b * n
    pf = pred.reshape(k, h, w)
    tf = target.reshape(k, h, w)
    mf = crop_masks.astype(jnp.float32).reshape(k, h, w)
    partial = pl.pallas_call(
        _lovasz_body,
        grid=(k,),
        in_specs=[
            pl.BlockSpec((1, h, w), lambda i: (i, 0, 0)),
            pl.BlockSpec((1, h, w), lambda i: (i, 0, 0)),
            pl.BlockSpec((1, h, w), lambda i: (i, 0, 0)),
        ],
        out_specs=pl.BlockSpec((1, 1, 1), lambda i: (i, 0, 0)),
        out_shape=jax.ShapeDtypeStruct((k, 1, 1), jnp.float32),
    )(pf, tf, mf)
    return jnp.sum(partial) / k


# speedup vs baseline: 3.9769x; 1.3403x over previous
import jax
import jax.numpy as jnp
from jax import lax
from jax.experimental import pallas as pl
from jax.experimental.pallas import tpu as pltpu

_H = 256
_W = 256
_NEG = -3.0e38


def _roll(x, sh, axis):
    n = x.shape[axis]
    sh %= n
    if sh == 0:
        return x
    if axis == 1:
        return jnp.concatenate([x[:, n - sh:], x[:, :n - sh]], axis=1)
    return jnp.concatenate([x[n - sh:, :], x[:n - sh, :]], axis=0)


def _lovasz_body(pred_ref, targ_ref, mask_ref, out_ref):
    p = pred_ref[0]
    t = targ_ref[0]
    m = mask_ref[0]

    row = lax.broadcasted_iota(jnp.int32, (_H, _W), 0)
    col = lax.broadcasted_iota(jnp.int32, (_H, _W), 1)

    signs = 2.0 * t - 1.0
    err = 1.0 - p * signs
    err = jnp.where(m > 0.0, err, _NEG)
    tm = t * m

    bits = lax.bitcast_convert_type(err, jnp.int32)
    key = jnp.where(bits < 0, bits ^ 0x7FFFFFFF, bits)
    key = (key & -2) | tm.astype(jnp.int32)

    lowc = [((col >> j) & 1) == 0 for j in range(8)]
    lowr = [((row >> j) & 1) == 0 for j in range(8)]
    for klog in range(1, 17):
        if klog < 8:
            asc = ~lowc[klog]
        elif klog < 16:
            asc = ~lowr[klog - 8]
        else:
            asc = None
        if asc is not None:
            flip = jnp.where(asc, -1, 0)
            key = key ^ flip
        for j in range(klog - 1, -1, -1):
            if j < 8:
                s, ax, low = 1 << j, 1, lowc[j]
            else:
                s, ax, low = 1 << (j - 8), 0, lowr[j - 8]
            key = jnp.where(low,
                            jnp.maximum(key, _roll(key, -s, ax)),
                            jnp.minimum(key, _roll(key, s, ax)))
        if asc is not None:
            key = key ^ flip

    gt = (key & 1).astype(jnp.float32)
    kb = key & -2
    ebits = jnp.where(kb < 0, kb ^ 0x7FFFFFFF, kb)
    es = lax.bitcast_convert_type(ebits, jnp.float32)
    relu = jnp.maximum(es, 0.0)

    cs = gt
    for sh in (1, 2, 4, 8, 16, 32, 64, 128):
        cs = cs + jnp.where(col >= sh, _roll(cs, sh, 1), 0.0)
    rowtot = jnp.broadcast_to(cs[:, _W - 1:_W], (_H, _W))
    ex = rowtot
    for sh in (1, 2, 4, 8, 16, 32, 64, 128):
        ex = ex + jnp.where(row >= sh, _roll(ex, sh, 0), 0.0)
    ex = ex - rowtot
    c_incl = cs + ex

    g_tot = jnp.sum(gt)
    jf = (row * _W + col + 1).astype(jnp.float32)
    jac = jf / (g_tot + jf - c_incl)

    nr = _roll(relu, -1, 1)
    ra0 = _roll(relu, -1, 0)
    col0 = jnp.broadcast_to(ra0[:, 0:1], (_H, _W))
    nr = jnp.where(col == _W - 1, col0, nr)
    nr = jnp.where((col == _W - 1) & (row == _H - 1), 0.0, nr)

    out_ref[0] = jnp.sum(jac * (relu - nr)).reshape(1, 1)


def kernel(pred, target, crop_masks):
    b, n, h, w = pred.shape
    k = b * n
    pf = pred.reshape(k, h, w)
    tf = target.reshape(k, h, w)
    mf = crop_masks.astype(jnp.float32).reshape(k, h, w)
    partial = pl.pallas_call(
        _lovasz_body,
        grid=(k,),
        in_specs=[
            pl.BlockSpec((1, h, w), lambda i: (i, 0, 0)),
            pl.BlockSpec((1, h, w), lambda i: (i, 0, 0)),
            pl.BlockSpec((1, h, w), lambda i: (i, 0, 0)),
        ],
        out_specs=pl.BlockSpec((1, 1, 1), lambda i: (i, 0, 0)),
        out_shape=jax.ShapeDtypeStruct((k, 1, 1), jnp.float32),
        compiler_params=pltpu.CompilerParams(
            dimension_semantics=("parallel",)),
    )(pf, tf, mf)
    return jnp.sum(partial) / k


# speedup vs baseline: 3.9870x; 1.0025x over previous
import jax
import jax.numpy as jnp
from jax import lax
from jax.experimental import pallas as pl
from jax.experimental.pallas import tpu as pltpu

_H = 256
_W = 256
_NEG = -3.0e38


def _roll(x, sh, axis):
    n = x.shape[axis]
    sh %= n
    if sh == 0:
        return x
    if axis == 1:
        return jnp.concatenate([x[:, n - sh:], x[:, :n - sh]], axis=1)
    return jnp.concatenate([x[n - sh:, :], x[:n - sh, :]], axis=0)


def _lovasz_body(pred_ref, targ_ref, mask_ref, out_ref):
    for u in range(2):
        out_ref[u] = _one_object(pred_ref[u], targ_ref[u],
                                 mask_ref[u]).reshape(1, 1)


def _one_object(p, t, m):
    row = lax.broadcasted_iota(jnp.int32, (_H, _W), 0)
    col = lax.broadcasted_iota(jnp.int32, (_H, _W), 1)

    signs = 2.0 * t - 1.0
    err = 1.0 - p * signs
    err = jnp.where(m > 0.0, err, _NEG)
    tm = t * m

    bits = lax.bitcast_convert_type(err, jnp.int32)
    key = jnp.where(bits < 0, bits ^ 0x7FFFFFFF, bits)
    key = (key & -2) | tm.astype(jnp.int32)

    lowc = [((col >> j) & 1) == 0 for j in range(8)]
    lowr = [((row >> j) & 1) == 0 for j in range(8)]
    for klog in range(1, 17):
        if klog < 8:
            asc = ~lowc[klog]
        elif klog < 16:
            asc = ~lowr[klog - 8]
        else:
            asc = None
        if asc is not None:
            flip = jnp.where(asc, -1, 0)
            key = key ^ flip
        for j in range(klog - 1, -1, -1):
            if j < 8:
                s, ax, low = 1 << j, 1, lowc[j]
            else:
                s, ax, low = 1 << (j - 8), 0, lowr[j - 8]
            key = jnp.where(low,
                            jnp.maximum(key, _roll(key, -s, ax)),
                            jnp.minimum(key, _roll(key, s, ax)))
        if asc is not None:
            key = key ^ flip

    gt = (key & 1).astype(jnp.float32)
    kb = key & -2
    ebits = jnp.where(kb < 0, kb ^ 0x7FFFFFFF, kb)
    es = lax.bitcast_convert_type(ebits, jnp.float32)
    relu = jnp.maximum(es, 0.0)

    cs = gt
    for sh in (1, 2, 4, 8, 16, 32, 64, 128):
        cs = cs + jnp.where(col >= sh, _roll(cs, sh, 1), 0.0)
    rowtot = jnp.broadcast_to(cs[:, _W - 1:_W], (_H, _W))
    ex = rowtot
    for sh in (1, 2, 4, 8, 16, 32, 64, 128):
        ex = ex + jnp.where(row >= sh, _roll(ex, sh, 0), 0.0)
    ex = ex - rowtot
    c_incl = cs + ex

    g_tot = jnp.sum(gt)
    jf = (row * _W + col + 1).astype(jnp.float32)
    jac = jf / (g_tot + jf - c_incl)

    nr = _roll(relu, -1, 1)
    ra0 = _roll(relu, -1, 0)
    col0 = jnp.broadcast_to(ra0[:, 0:1], (_H, _W))
    nr = jnp.where(col == _W - 1, col0, nr)
    nr = jnp.where((col == _W - 1) & (row == _H - 1), 0.0, nr)

    return jnp.sum(jac * (relu - nr))


def kernel(pred, target, crop_masks):
    b, n, h, w = pred.shape
    k = b * n
    pf = pred.reshape(k, h, w)
    tf = target.reshape(k, h, w)
    mf = crop_masks.astype(jnp.float32).reshape(k, h, w)
    partial = pl.pallas_call(
        _lovasz_body,
        grid=(k // 2,),
        in_specs=[
            pl.BlockSpec((2, h, w), lambda i: (i, 0, 0)),
            pl.BlockSpec((2, h, w), lambda i: (i, 0, 0)),
            pl.BlockSpec((2, h, w), lambda i: (i, 0, 0)),
        ],
        out_specs=pl.BlockSpec((2, 1, 1), lambda i: (i, 0, 0)),
        out_shape=jax.ShapeDtypeStruct((k, 1, 1), jnp.float32),
        compiler_params=pltpu.CompilerParams(
            dimension_semantics=("parallel",)),
    )(pf, tf, mf)
    return jnp.sum(partial) / k


# speedup vs baseline: 5.4677x; 1.3714x over previous
import jax
import jax.numpy as jnp
from jax import lax
from jax.experimental import pallas as pl
from jax.experimental.pallas import tpu as pltpu

_H = 256
_W = 256
_NEG = -3.0e38


def _roll(x, sh, axis):
    n = x.shape[axis]
    sh %= n
    if sh == 0:
        return x
    if axis == 1:
        return jnp.concatenate([x[:, n - sh:], x[:, :n - sh]], axis=1)
    return jnp.concatenate([x[n - sh:, :], x[:n - sh, :]], axis=0)


def _lovasz_body(pred_ref, targ_ref, mask_ref, out_ref):
    for u in range(2):
        out_ref[u] = _one_object(pred_ref[u], targ_ref[u],
                                 mask_ref[u]).reshape(1, 1)


def _one_object(p, t, m):
    row = lax.broadcasted_iota(jnp.int32, (_H, _W), 0)
    col = lax.broadcasted_iota(jnp.int32, (_H, _W), 1)

    signs = 2.0 * t - 1.0
    err = 1.0 - p * signs
    err = jnp.where(m > 0.0, err, _NEG)
    tm = t * m

    bits = lax.bitcast_convert_type(err, jnp.int32)
    key = jnp.where(bits < 0, bits ^ 0x7FFFFFFF, bits)
    key = (key & -2) | tm.astype(jnp.int32)

    lowc = [((col >> j) & 1) == 0 for j in range(8)]
    lowr = [((row >> j) & 1) == 0 for j in range(8)]
    for klog in range(1, 17):
        if klog < 8:
            asc = ~lowr[klog]
        elif klog < 16:
            asc = ~lowc[klog - 8]
        else:
            asc = None
        if asc is not None:
            flip = jnp.where(asc, -1, 0)
            key = key ^ flip
        for j in range(klog - 1, -1, -1):
            if j < 8:
                s, ax, low = 1 << j, 0, lowr[j]
            else:
                s, ax, low = 1 << (j - 8), 1, lowc[j - 8]
            key = jnp.where(low,
                            jnp.maximum(key, _roll(key, -s, ax)),
                            jnp.minimum(key, _roll(key, s, ax)))
        if asc is not None:
            key = key ^ flip

    gt = (key & 1).astype(jnp.float32)
    kb = key & -2
    ebits = jnp.where(kb < 0, kb ^ 0x7FFFFFFF, kb)
    es = lax.bitcast_convert_type(ebits, jnp.float32)
    relu = jnp.maximum(es, 0.0)

    cs = gt
    for sh in (1, 2, 4, 8, 16, 32, 64, 128):
        cs = cs + jnp.where(row >= sh, _roll(cs, sh, 0), 0.0)
    coltot = jnp.broadcast_to(cs[_H - 1:_H, :], (_H, _W))
    ex = coltot
    for sh in (1, 2, 4, 8, 16, 32, 64, 128):
        ex = ex + jnp.where(col >= sh, _roll(ex, sh, 1), 0.0)
    ex = ex - coltot
    c_incl = cs + ex

    g_tot = jnp.sum(gt)
    jf = (col * _H + row + 1).astype(jnp.float32)
    jac = jf / (g_tot + jf - c_incl)

    nr = _roll(relu, -1, 0)
    rc0 = _roll(relu, -1, 1)
    row0 = jnp.broadcast_to(rc0[0:1, :], (_H, _W))
    nr = jnp.where(row == _H - 1, row0, nr)
    nr = jnp.where((row == _H - 1) & (col == _W - 1), 0.0, nr)

    return jnp.sum(jac * (relu - nr))


def kernel(pred, target, crop_masks):
    b, n, h, w = pred.shape
    k = b * n
    pf = pred.reshape(k, h, w)
    tf = target.reshape(k, h, w)
    mf = crop_masks.astype(jnp.float32).reshape(k, h, w)
    partial = pl.pallas_call(
        _lovasz_body,
        grid=(k // 2,),
        in_specs=[
            pl.BlockSpec((2, h, w), lambda i: (i, 0, 0)),
            pl.BlockSpec((2, h, w), lambda i: (i, 0, 0)),
            pl.BlockSpec((2, h, w), lambda i: (i, 0, 0)),
        ],
        out_specs=pl.BlockSpec((2, 1, 1), lambda i: (i, 0, 0)),
        out_shape=jax.ShapeDtypeStruct((k, 1, 1), jnp.float32),
        compiler_params=pltpu.CompilerParams(
            dimension_semantics=("parallel",)),
    )(pf, tf, mf)
    return jnp.sum(partial) / k
